# baseline (speedup 1.0000x reference)
import functools

import jax
import jax.numpy as jnp
from jax import lax
from jax.experimental import pallas as pl
from jax.experimental.pallas import tpu as pltpu
from jax.experimental.pallas import tpu_sc as plsc

B = 16384
L = 200
DIM = 32
NW = 32
BPW = B // NW
NBT = BPW // 128


@jax.jit
def _sc_gather_t(table, xt):
    mesh = plsc.VectorSubcoreMesh(core_axis_name="c", subcore_axis_name="s")

    @functools.partial(
        pl.kernel,
        mesh=mesh,
        out_type=jax.ShapeDtypeStruct((L, DIM // 8, B // 128, 1024),
                                      jnp.float32),
        scratch_types=[
            pltpu.VMEM((BPW,), jnp.int32),
            pltpu.VMEM((BPW,), jnp.int32),
            pltpu.VMEM((BPW, DIM), jnp.float32),
            pltpu.VMEM((BPW, DIM), jnp.float32),
            pltpu.VMEM((16, 1024), jnp.float32),
            pltpu.VMEM((16, 1024), jnp.float32),
            pltpu.SemaphoreType.DMA,
            pltpu.SemaphoreType.DMA,
            pltpu.SemaphoreType.DMA,
            pltpu.SemaphoreType.DMA,
            pltpu.SemaphoreType.DMA,
            pltpu.SemaphoreType.DMA,
        ],
        compiler_params=pltpu.CompilerParams(use_tc_tiling_on_sc=False,
                                             needs_layout_passes=False),
    )
    def k(table_hbm, xt_hbm, out_hbm, ibuf0, ibuf1, rbuf0, rbuf1,
          tbuf0, tbuf1, isem0, isem1, gsem0, gsem1, osem0, osem1):
        wid = lax.axis_index("s") * 2 + lax.axis_index("c")
        wb0 = wid * BPW
        wbt0 = wid * NBT

        iota = lax.iota(jnp.int32, 16)
        iota_shr3 = iota >> 3
        iota_a7_128 = (iota & 7) * 128

        def idx_start(l, ibuf, isem):
            pltpu.async_copy(xt_hbm.at[l, pl.ds(wb0, BPW)], ibuf, isem)

        def idx_wait(ibuf, isem):
            pltpu.make_async_copy(xt_hbm.at[0, pl.ds(wb0, BPW)],
                                  ibuf, isem).wait()

        def gat_start(ibuf, rbuf, gsem):
            pltpu.async_copy(table_hbm.at[ibuf], rbuf, gsem)

        def gat_wait(ibuf, rbuf, gsem):
            pltpu.make_async_copy(table_hbm.at[ibuf], rbuf, gsem).wait()

        def out_start(l, tbuf, osem):
            for bt in range(NBT):
                pltpu.async_copy(tbuf.at[pl.ds(bt * 4, 4), :],
                                 out_hbm.at[l, :, wbt0 + bt], osem)

        def out_wait(tbuf, osem):
            for bt in range(NBT):
                pltpu.make_async_copy(tbuf.at[pl.ds(bt * 4, 4), :],
                                      out_hbm.at[0, :, wbt0 + bt],
                                      osem).wait()

        def transpose(rbuf, tbuf):
            def body(q, carry):
                for u in range(4):
                    r = 4 * q + u
                    bt = r >> 7
                    col = iota_a7_128 + (r & 127)
                    row = iota_shr3 + (bt * 4)
                    plsc.store_scatter(tbuf, [row, col],
                                       rbuf[r, pl.ds(0, 16)])
                    plsc.store_scatter(tbuf, [row + 2, col],
                                       rbuf[r, pl.ds(16, 16)])
                return carry

            lax.fori_loop(0, BPW // 4, body, 0)

        idx_start(0, ibuf0, isem0)
        idx_start(1, ibuf1, isem1)
        idx_wait(ibuf0, isem0)
        gat_start(ibuf0, rbuf0, gsem0)

        def pair(i2, carry):
            a = 2 * i2
            b = a + 1
            idx_wait(ibuf1, isem1)
            gat_start(ibuf1, rbuf1, gsem1)
            gat_wait(ibuf0, rbuf0, gsem0)

            @pl.when(a + 2 < L)
            def _():
                idx_start(a + 2, ibuf0, isem0)

            @pl.when(i2 >= 1)
            def _():
                out_wait(tbuf0, osem0)

            transpose(rbuf0, tbuf0)
            out_start(a, tbuf0, osem0)
            gat_wait(ibuf1, rbuf1, gsem1)

            @pl.when(b + 2 < L)
            def _():
                idx_start(b + 2, ibuf1, isem1)

            @pl.when(i2 >= 1)
            def _():
                out_wait(tbuf1, osem1)

            transpose(rbuf1, tbuf1)
            out_start(b, tbuf1, osem1)

            @pl.when(a + 2 < L)
            def _():
                idx_wait(ibuf0, isem0)
                gat_start(ibuf0, rbuf0, gsem0)

            return carry

        lax.fori_loop(0, L // 2, pair, 0)
        out_wait(tbuf0, osem0)
        out_wait(tbuf1, osem1)

    return k(table, xt)


def kernel(x, table):
    xt = jnp.swapaxes(x, 0, 1)
    out_t = _sc_gather_t(table, xt)
    out5 = out_t.reshape(L, DIM // 8, B // 128, 8, 128)
    return out5.transpose(2, 4, 0, 1, 3).reshape(B, L, DIM)


# speedup vs baseline: 1.5827x; 1.5827x over previous
import functools

import jax
import jax.numpy as jnp
from jax import lax
from jax.experimental import pallas as pl
from jax.experimental.pallas import tpu as pltpu
from jax.experimental.pallas import tpu_sc as plsc

B = 16384
L = 200
DIM = 32
NW = 32
BPW = B // NW
NBT = BPW // 128


@jax.jit
def _sc_gather_t(table, xt):
    mesh = plsc.VectorSubcoreMesh(core_axis_name="c", subcore_axis_name="s")

    @functools.partial(
        pl.kernel,
        mesh=mesh,
        out_type=jax.ShapeDtypeStruct((L, DIM // 8, B // 128, 1024),
                                      jnp.float32),
        scratch_types=[
            pltpu.VMEM((BPW,), jnp.int32),
            pltpu.VMEM((BPW,), jnp.int32),
            pltpu.VMEM((BPW, DIM + 1), jnp.float32),
            pltpu.VMEM((BPW, DIM + 1), jnp.float32),
            pltpu.VMEM((16, 1024), jnp.float32),
            pltpu.VMEM((16, 1024), jnp.float32),
            pltpu.SemaphoreType.DMA,
            pltpu.SemaphoreType.DMA,
            pltpu.SemaphoreType.DMA,
            pltpu.SemaphoreType.DMA,
            pltpu.SemaphoreType.DMA,
            pltpu.SemaphoreType.DMA,
        ],
        compiler_params=pltpu.CompilerParams(use_tc_tiling_on_sc=False,
                                             needs_layout_passes=False),
    )
    def k(table_hbm, xt_hbm, out_hbm, ibuf0, ibuf1, rbuf0, rbuf1,
          tbuf0, tbuf1, isem0, isem1, gsem0, gsem1, osem0, osem1):
        wid = lax.axis_index("s") * 2 + lax.axis_index("c")
        wb0 = wid * BPW
        wbt0 = wid * NBT

        iota = lax.iota(jnp.int32, 16)

        def idx_start(l, ibuf, isem):
            pltpu.async_copy(xt_hbm.at[l, pl.ds(wb0, BPW)], ibuf, isem)

        def idx_wait(ibuf, isem):
            pltpu.make_async_copy(xt_hbm.at[0, pl.ds(wb0, BPW)],
                                  ibuf, isem).wait()

        def gat_start(ibuf, rbuf, gsem):
            pltpu.async_copy(table_hbm.at[ibuf], rbuf, gsem)

        def gat_wait(ibuf, rbuf, gsem):
            pltpu.make_async_copy(table_hbm.at[ibuf], rbuf, gsem).wait()

        def out_start(l, tbuf, osem):
            for bt in range(NBT):
                pltpu.async_copy(tbuf.at[pl.ds(bt * 4, 4), :],
                                 out_hbm.at[l, :, wbt0 + bt], osem)

        def out_wait(tbuf, osem):
            for bt in range(NBT):
                pltpu.make_async_copy(tbuf.at[pl.ds(bt * 4, 4), :],
                                      out_hbm.at[0, :, wbt0 + bt],
                                      osem).wait()

        def transpose(rbuf, tbuf):
            for bt in range(NBT):
                rowidx = [iota + (bt * 128 + 16 * j) for j in range(8)]

                def body(d, carry, rowidx=rowidx, bt=bt):
                    colv = lax.full((16,), d, jnp.int32)
                    trow = bt * 4 + (d >> 3)
                    tcol = (d & 7) * 128
                    for j in range(8):
                        v = plsc.load_gather(rbuf, [rowidx[j], colv])
                        tbuf[trow, pl.ds(tcol + 16 * j, 16)] = v
                    return carry

                lax.fori_loop(0, DIM, body, 0)

        idx_start(0, ibuf0, isem0)
        idx_start(1, ibuf1, isem1)
        idx_wait(ibuf0, isem0)
        gat_start(ibuf0, rbuf0, gsem0)

        def pair(i2, carry):
            a = 2 * i2
            b = a + 1
            idx_wait(ibuf1, isem1)
            gat_start(ibuf1, rbuf1, gsem1)
            gat_wait(ibuf0, rbuf0, gsem0)

            @pl.when(a + 2 < L)
            def _():
                idx_start(a + 2, ibuf0, isem0)

            @pl.when(i2 >= 1)
            def _():
                out_wait(tbuf0, osem0)

            transpose(rbuf0, tbuf0)
            out_start(a, tbuf0, osem0)
            gat_wait(ibuf1, rbuf1, gsem1)

            @pl.when(b + 2 < L)
            def _():
                idx_start(b + 2, ibuf1, isem1)

            @pl.when(i2 >= 1)
            def _():
                out_wait(tbuf1, osem1)

            transpose(rbuf1, tbuf1)
            out_start(b, tbuf1, osem1)

            @pl.when(a + 2 < L)
            def _():
                idx_wait(ibuf0, isem0)
                gat_start(ibuf0, rbuf0, gsem0)

            return carry

        lax.fori_loop(0, L // 2, pair, 0)
        out_wait(tbuf0, osem0)
        out_wait(tbuf1, osem1)

    return k(table, xt)


def kernel(x, table):
    xt = jnp.swapaxes(x, 0, 1)
    tablep = jnp.pad(table, ((0, 0), (0, 1)))
    out_t = _sc_gather_t(tablep, xt)
    out5 = out_t.reshape(L, DIM // 8, B // 128, 8, 128)
    return out5.transpose(2, 4, 0, 1, 3).reshape(B, L, DIM)


# speedup vs baseline: 2.6743x; 1.6897x over previous
import functools

import jax
import jax.numpy as jnp
from jax import lax
from jax.experimental import pallas as pl
from jax.experimental.pallas import tpu as pltpu
from jax.experimental.pallas import tpu_sc as plsc

B = 16384
L = 200
DIM = 32
PITCH = DIM + 1
NW = 32
BPW = B // NW
NBT = BPW // 128


@jax.jit
def _sc_gather_t(table, xt):
    mesh = plsc.VectorSubcoreMesh(core_axis_name="c", subcore_axis_name="s")

    @functools.partial(
        pl.kernel,
        mesh=mesh,
        out_type=jax.ShapeDtypeStruct((L, DIM // 8, B // 128, 1024),
                                      jnp.float32),
        scratch_types=[
            pltpu.VMEM((BPW,), jnp.int32),
            pltpu.VMEM((BPW,), jnp.int32),
            pltpu.VMEM((BPW, DIM), jnp.float32),
            pltpu.VMEM((BPW, DIM), jnp.float32),
            pltpu.VMEM((BPW, PITCH), jnp.float32),
            pltpu.VMEM((BPW, PITCH), jnp.float32),
            pltpu.VMEM((16, 1024), jnp.float32),
            pltpu.VMEM((16, 1024), jnp.float32),
            pltpu.SemaphoreType.DMA,
            pltpu.SemaphoreType.DMA,
            pltpu.SemaphoreType.DMA,
            pltpu.SemaphoreType.DMA,
            pltpu.SemaphoreType.DMA,
            pltpu.SemaphoreType.DMA,
        ],
        compiler_params=pltpu.CompilerParams(use_tc_tiling_on_sc=False,
                                             needs_layout_passes=False),
    )
    def k(table_hbm, xt_hbm, out_hbm, ibuf0, ibuf1, rbuf0, rbuf1,
          sbuf0, sbuf1, tbuf0, tbuf1,
          isem0, isem1, gsem0, gsem1, osem0, osem1):
        wid = lax.axis_index("s") * 2 + lax.axis_index("c")
        wb0 = wid * BPW
        wbt0 = wid * NBT

        iota = lax.iota(jnp.int32, 16)

        def idx_start(l, ibuf, isem):
            pltpu.async_copy(xt_hbm.at[l, pl.ds(wb0, BPW)], ibuf, isem)

        def idx_wait(ibuf, isem):
            pltpu.make_async_copy(xt_hbm.at[0, pl.ds(wb0, BPW)],
                                  ibuf, isem).wait()

        def gat_start(ibuf, rbuf, gsem):
            pltpu.async_copy(table_hbm.at[ibuf], rbuf, gsem)

        def gat_wait(ibuf, rbuf, gsem):
            pltpu.make_async_copy(table_hbm.at[ibuf], rbuf, gsem).wait()

        def repitch(rbuf, sbuf):
            def body(q, carry):
                vs = []
                for u in range(4):
                    r = 4 * q + u
                    vs.append((r, rbuf[r, pl.ds(0, 16)],
                               rbuf[r, pl.ds(16, 16)]))
                for r, v0, v1 in vs:
                    sbuf[r, pl.ds(0, 16)] = v0
                    sbuf[r, pl.ds(16, 16)] = v1
                return carry

            lax.fori_loop(0, BPW // 4, body, 0)

        def out_start(l, tbuf, osem):
            for bt in range(NBT):
                pltpu.async_copy(tbuf.at[pl.ds(bt * 4, 4), :],
                                 out_hbm.at[l, :, wbt0 + bt], osem)

        def out_wait(tbuf, osem):
            for bt in range(NBT):
                pltpu.make_async_copy(tbuf.at[pl.ds(bt * 4, 4), :],
                                      out_hbm.at[0, :, wbt0 + bt],
                                      osem).wait()

        def transpose(sbuf, tbuf):
            for bt in range(NBT):
                rowidx = [iota + (bt * 128 + 16 * j) for j in range(8)]

                def body(d, carry, rowidx=rowidx, bt=bt):
                    colv = lax.full((16,), d, jnp.int32)
                    trow = bt * 4 + (d >> 3)
                    tcol = (d & 7) * 128
                    vs = [plsc.load_gather(sbuf, [rowidx[j], colv])
                          for j in range(8)]
                    for j in range(8):
                        tbuf[trow, pl.ds(tcol + 16 * j, 16)] = vs[j]
                    return carry

                lax.fori_loop(0, DIM, body, 0)

        idx_start(0, ibuf0, isem0)
        idx_start(1, ibuf1, isem1)
        idx_wait(ibuf0, isem0)
        gat_start(ibuf0, rbuf0, gsem0)

        def pair(i2, carry):
            a = 2 * i2
            b = a + 1
            idx_wait(ibuf1, isem1)
            gat_start(ibuf1, rbuf1, gsem1)
            gat_wait(ibuf0, rbuf0, gsem0)

            @pl.when(a + 2 < L)
            def _():
                idx_start(a + 2, ibuf0, isem0)

            repitch(rbuf0, sbuf0)

            @pl.when(i2 >= 1)
            def _():
                out_wait(tbuf0, osem0)

            transpose(sbuf0, tbuf0)
            out_start(a, tbuf0, osem0)
            gat_wait(ibuf1, rbuf1, gsem1)

            @pl.when(b + 2 < L)
            def _():
                idx_start(b + 2, ibuf1, isem1)

            repitch(rbuf1, sbuf1)

            @pl.when(a + 2 < L)
            def _():
                idx_wait(ibuf0, isem0)
                gat_start(ibuf0, rbuf0, gsem0)

            @pl.when(i2 >= 1)
            def _():
                out_wait(tbuf1, osem1)

            transpose(sbuf1, tbuf1)
            out_start(b, tbuf1, osem1)

            return carry

        lax.fori_loop(0, L // 2, pair, 0)
        out_wait(tbuf0, osem0)
        out_wait(tbuf1, osem1)

    return k(table, xt)


def kernel(x, table):
    xt = jnp.swapaxes(x, 0, 1)
    out_t = _sc_gather_t(table, xt)
    out5 = out_t.reshape(L, DIM // 8, B // 128, 8, 128)
    return out5.transpose(2, 4, 0, 1, 3).reshape(B, L, DIM)
